# baseline (speedup 1.0000x reference)
import jax
import jax.numpy as jnp
from jax.experimental import pallas as pl
from jax.experimental.pallas import tpu as pltpu

_CH = 32
_NBUF = 8


def _body(a_hbm, f_ref, o_ref, buf, sems):
    B, M, K = a_hbm.shape
    cpb = M // _CH
    total = B * cpb
    ngroups = total // _NBUF

    def copy(c, slot):
        b = c // cpb
        r = jax.lax.rem(c, cpb)
        return pltpu.make_async_copy(
            a_hbm.at[b, pl.ds(r * _CH, _CH), :],
            buf.at[slot],
            sems.at[slot],
        )

    for slot in range(_NBUF):
        copy(slot, slot).start()

    def group(g, carry):
        base = g * _NBUF
        for slot in range(_NBUF):
            c = base + slot
            copy(c, slot).wait()
            o_ref[pl.ds(c * _CH, _CH), :] = buf[slot, :, :64]

            @pl.when(c + _NBUF < total)
            def _(c=c, slot=slot):
                copy(c + _NBUF, slot).start()

        return carry

    jax.lax.fori_loop(0, ngroups, group, 0)


def kernel(features, A):
    B, M, K = A.shape
    N = features.shape[-1]
    A = A.reshape(B, M // 8, K * 8)
    out_flat = pl.pallas_call(
        _body,
        in_specs=[
            pl.BlockSpec(memory_space=pltpu.MemorySpace.HBM),
            pl.BlockSpec(memory_space=pltpu.MemorySpace.VMEM),
        ],
        out_specs=pl.BlockSpec(memory_space=pltpu.MemorySpace.VMEM),
        out_shape=jax.ShapeDtypeStruct((B * M // 8, N), jnp.float32),
        scratch_shapes=[
            pltpu.VMEM((_NBUF, _CH, K * 8), jnp.float32),
            pltpu.SemaphoreType.DMA((_NBUF,)),
        ],
    )(A, features)
    out = jnp.broadcast_to(out_flat.reshape(B, M // 8, 1, N), (B, M // 8, 8, N))
    return out.reshape(B, M, N)


# speedup vs baseline: 3.4499x; 3.4499x over previous
import jax
import jax.numpy as jnp
from jax.experimental import pallas as pl
from jax.experimental.pallas import tpu as pltpu


def _bmm_kernel(f_ref, a_ref, o_ref):
    o_ref[0] = jnp.dot(a_ref[0], f_ref[0], preferred_element_type=jnp.float32)


def kernel(features, A):
    B, M, K = A.shape
    N = features.shape[-1]
    return pl.pallas_call(
        _bmm_kernel,
        grid=(B,),
        in_specs=[
            pl.BlockSpec((1, K, N), lambda b: (b, 0, 0)),
            pl.BlockSpec((1, M, K), lambda b: (b, 0, 0)),
        ],
        out_specs=pl.BlockSpec((1, M, N), lambda b: (b, 0, 0)),
        out_shape=jax.ShapeDtypeStruct((B, M, N), jnp.float32),
        compiler_params=pltpu.CompilerParams(
            dimension_semantics=("arbitrary",),
        ),
    )(features, A)


# speedup vs baseline: 3.4526x; 1.0008x over previous
import jax
import jax.numpy as jnp
from jax.experimental import pallas as pl
from jax.experimental.pallas import tpu as pltpu


def _bmm_kernel(f_ref, a_ref, o_ref, a16, f16):
    a16[...] = a_ref[0].astype(jnp.bfloat16)
    f16[...] = f_ref[0].astype(jnp.bfloat16)
    o_ref[0] = jnp.dot(a16[...], f16[...], preferred_element_type=jnp.float32)


def kernel(features, A):
    B, M, K = A.shape
    N = features.shape[-1]
    return pl.pallas_call(
        _bmm_kernel,
        grid=(B,),
        in_specs=[
            pl.BlockSpec((1, K, N), lambda b: (b, 0, 0)),
            pl.BlockSpec((1, M, K), lambda b: (b, 0, 0)),
        ],
        out_specs=pl.BlockSpec((1, M, N), lambda b: (b, 0, 0)),
        out_shape=jax.ShapeDtypeStruct((B, M, N), jnp.float32),
        scratch_shapes=[
            pltpu.VMEM((M, K), jnp.bfloat16),
            pltpu.VMEM((K, N), jnp.bfloat16),
        ],
        compiler_params=pltpu.CompilerParams(
            dimension_semantics=("arbitrary",),
        ),
    )(features, A)
